# baseline (device time: 57478 ns/iter reference)
import jax
import jax.numpy as jnp
from jax import lax
from jax.experimental import pallas as pl
from jax.experimental.pallas import tpu as pltpu

N_DEV = 4
T = 2048
D = 1024
V_SHARD = 16384
HS = T // 4
QS = T // 8
NCB = 128


def _body(absrow_sref, vid_sref, cnt_sref, E_ref, maskf_ref, out_ref,
          stage_ref, acc_ref, rbuf_ref, gsems, send_sems, recv_sems):
    my = lax.axis_index("i")
    p1 = my ^ 1
    p2 = 3 - my

    def gather_block(b, c):
        k_from = b * NCB

        def step(j, carry):
            k = k_from + j
            pltpu.make_async_copy(
                E_ref.at[pl.ds(vid_sref[k], 1)],
                stage_ref.at[pl.ds(absrow_sref[k], 1)],
                gsems.at[c],
            ).start()
            return carry

        lax.fori_loop(0, cnt_sref[b], step, jnp.int32(0))

    def drain_convert(b, c):
        def wait_one(_, carry):
            pltpu.make_async_copy(
                E_ref.at[pl.ds(0, 1)], stage_ref.at[pl.ds(0, 1)], gsems.at[c]
            ).wait()
            return carry

        lax.fori_loop(0, cnt_sref[b], wait_one, jnp.int32(0))
        sl = pl.ds(b * QS, QS)
        acc_ref[sl] = jnp.where(
            maskf_ref[sl] > 0, stage_ref[sl], 0.0
        ).astype(jnp.bfloat16)

    hA = (my ^ (my >> 1)) & 1
    qA = my >> 1
    hB = my >> 1
    qB = my & 1

    khA = hA * HS
    shA = (1 - hA) * HS
    kqA = khA + qA * QS
    sqA = khA + (1 - qA) * QS
    khB = 2 * HS + hB * HS
    shB = 2 * HS + (1 - hB) * HS
    kqB = khB + qB * QS
    sqB = khB + (1 - qB) * QS

    def exch(sem_idx, src_off, n_rows, rbuf_off, peer):
        return pltpu.make_async_remote_copy(
            src_ref=acc_ref.at[pl.ds(src_off, n_rows)],
            dst_ref=rbuf_ref.at[pl.ds(rbuf_off, n_rows)],
            send_sem=send_sems.at[sem_idx], recv_sem=recv_sems.at[sem_idx],
            device_id=(peer,), device_id_type=pl.DeviceIdType.MESH,
        )

    bsA = shA // QS
    bsB = shB // QS
    bkA = khA // QS
    bkB = khB // QS

    gather_block(bsA, 0)
    gather_block(bsA + 1, 1)

    barrier = pltpu.get_barrier_semaphore()
    for nbr in (p1, p2):
        pl.semaphore_signal(
            barrier, inc=1, device_id=(nbr,),
            device_id_type=pl.DeviceIdType.MESH,
        )
    pl.semaphore_wait(barrier, 2)

    drain_convert(bsA, 0)
    a1a = exch(0, shA, QS, 0, p1)
    a1a.start()

    gather_block(bsB, 2)
    drain_convert(bsA + 1, 1)
    a1b = exch(1, shA + QS, QS, QS, p1)
    a1b.start()

    gather_block(bsB + 1, 3)
    drain_convert(bsB, 2)
    b1a = exch(2, shB, QS, HS, p2)
    b1a.start()

    gather_block(bkA, 4)
    drain_convert(bsB + 1, 3)
    b1b = exch(3, shB + QS, QS, HS + QS, p2)
    b1b.start()

    gather_block(bkA + 1, 5)
    drain_convert(bkA, 4)
    drain_convert(bkA + 1, 5)

    a1a.wait()
    acc_ref[pl.ds(khA, QS)] = acc_ref[pl.ds(khA, QS)] + rbuf_ref[pl.ds(0, QS)]
    a1b.wait()
    acc_ref[pl.ds(khA + QS, QS)] = (
        acc_ref[pl.ds(khA + QS, QS)] + rbuf_ref[pl.ds(QS, QS)]
    )
    a2 = exch(4, sqA, QS, 2 * HS, p2)
    a2.start()

    gather_block(bkB, 6)
    gather_block(bkB + 1, 7)
    drain_convert(bkB, 6)
    drain_convert(bkB + 1, 7)

    b1a.wait()
    acc_ref[pl.ds(khB, QS)] = acc_ref[pl.ds(khB, QS)] + rbuf_ref[pl.ds(HS, QS)]
    b1b.wait()
    acc_ref[pl.ds(khB + QS, QS)] = (
        acc_ref[pl.ds(khB + QS, QS)] + rbuf_ref[pl.ds(HS + QS, QS)]
    )
    b2 = exch(5, sqB, QS, 2 * HS + QS, p1)
    b2.start()

    a2.wait()
    acc_ref[pl.ds(kqA, QS)] = (
        acc_ref[pl.ds(kqA, QS)] + rbuf_ref[pl.ds(2 * HS, QS)]
    )
    a3 = exch(6, kqA, QS, 3 * HS, p2)
    a3.start()
    a4a = exch(8, kqA, QS, 4 * HS, p1)
    a4a.start()

    b2.wait()
    acc_ref[pl.ds(kqB, QS)] = (
        acc_ref[pl.ds(kqB, QS)] + rbuf_ref[pl.ds(2 * HS + QS, QS)]
    )
    b3 = exch(7, kqB, QS, 3 * HS + QS, p1)
    b3.start()
    b4a = exch(10, kqB, QS, 5 * HS, p2)
    b4a.start()

    a3.wait()
    acc_ref[pl.ds(sqA, QS)] = rbuf_ref[pl.ds(3 * HS, QS)]
    a4b = exch(9, sqA, QS, 4 * HS + QS, p1)
    a4b.start()

    b3.wait()
    acc_ref[pl.ds(sqB, QS)] = rbuf_ref[pl.ds(3 * HS + QS, QS)]
    b4b = exch(11, sqB, QS, 5 * HS + QS, p2)
    b4b.start()

    out_ref[pl.ds(khA, HS)] = acc_ref[pl.ds(khA, HS)].astype(jnp.float32)
    out_ref[pl.ds(khB, HS)] = acc_ref[pl.ds(khB, HS)].astype(jnp.float32)

    a4a.wait()
    out_ref[pl.ds(shA + qA * QS, QS)] = (
        rbuf_ref[pl.ds(4 * HS, QS)].astype(jnp.float32)
    )
    a4b.wait()
    out_ref[pl.ds(shA + (1 - qA) * QS, QS)] = (
        rbuf_ref[pl.ds(4 * HS + QS, QS)].astype(jnp.float32)
    )
    b4a.wait()
    out_ref[pl.ds(shB + (1 - qB) * QS, QS)] = (
        rbuf_ref[pl.ds(5 * HS, QS)].astype(jnp.float32)
    )
    b4b.wait()
    out_ref[pl.ds(shB + qB * QS, QS)] = (
        rbuf_ref[pl.ds(5 * HS + QS, QS)].astype(jnp.float32)
    )


def kernel(ids, E):
    my = lax.axis_index("i")
    loc = (ids - my * V_SHARD).astype(jnp.int32)
    owned = (loc >= 0) & (loc < V_SHARD)
    maskf = owned.astype(jnp.float32)[:, None]

    owned_b = owned.reshape(8, QS)
    loc_b = jnp.clip(loc, 0, V_SHARD - 1).reshape(8, QS)
    row_b = jnp.arange(T, dtype=jnp.int32).reshape(8, QS)
    cnt = owned_b.sum(axis=1).astype(jnp.int32)

    cs = jnp.cumsum(owned_b.astype(jnp.int32), axis=1)
    k = jnp.arange(NCB, dtype=jnp.int32)[None, :, None]
    M = ((cs[:, None, :] == (k + 1)) & owned_b[:, None, :]).astype(jnp.float32)
    R = jnp.stack([row_b, loc_b], axis=-1).astype(jnp.float32)
    pairs = jnp.matmul(M, R, precision=lax.Precision.HIGHEST).astype(jnp.int32)
    absrow = pairs[..., 0].reshape(-1)
    vid = pairs[..., 1].reshape(-1)

    grid_spec = pltpu.PrefetchScalarGridSpec(
        num_scalar_prefetch=3,
        grid=(1,),
        in_specs=[
            pl.BlockSpec(memory_space=pl.ANY),
            pl.BlockSpec(memory_space=pltpu.VMEM),
        ],
        out_specs=pl.BlockSpec(memory_space=pltpu.VMEM),
        scratch_shapes=[
            pltpu.VMEM((T, D), jnp.float32),
            pltpu.VMEM((T, D), jnp.bfloat16),
            pltpu.VMEM((6 * HS, D), jnp.bfloat16),
            pltpu.SemaphoreType.DMA((8,)),
            pltpu.SemaphoreType.DMA((12,)),
            pltpu.SemaphoreType.DMA((12,)),
        ],
    )
    return pl.pallas_call(
        _body,
        grid_spec=grid_spec,
        out_shape=jax.ShapeDtypeStruct((T, D), jnp.float32),
        compiler_params=pltpu.CompilerParams(collective_id=0),
    )(absrow, vid, cnt, E, maskf)


# device time: 56691 ns/iter; 1.0139x vs baseline; 1.0139x over previous
import jax
import jax.numpy as jnp
from jax import lax
from jax.experimental import pallas as pl
from jax.experimental.pallas import tpu as pltpu

N_DEV = 4
T = 2048
D = 1024
V_SHARD = 16384
HS = T // 4
QS = T // 8
NCB = 128


def _body(absrow_sref, vid_sref, cnt_sref, E_ref, maskf_ref, out_ref,
          stage_ref, acc_ref, rbuf_ref, gsems, send_sems, recv_sems):
    my = lax.axis_index("i")
    p1 = my ^ 1
    p2 = 3 - my

    barrier = pltpu.get_barrier_semaphore()
    for nbr in (p1, p2):
        pl.semaphore_signal(
            barrier, inc=1, device_id=(nbr,),
            device_id_type=pl.DeviceIdType.MESH,
        )
    pl.semaphore_wait(barrier, 2)

    def gather_block(b, c):
        k_from = b * NCB

        def step(j, carry):
            k = k_from + j
            pltpu.make_async_copy(
                E_ref.at[pl.ds(vid_sref[k], 1)],
                stage_ref.at[pl.ds(absrow_sref[k], 1)],
                gsems.at[c],
            ).start()
            return carry

        lax.fori_loop(0, cnt_sref[b], step, jnp.int32(0))

    def drain_convert(b, c):
        def wait_one(_, carry):
            pltpu.make_async_copy(
                E_ref.at[pl.ds(0, 1)], stage_ref.at[pl.ds(0, 1)], gsems.at[c]
            ).wait()
            return carry

        lax.fori_loop(0, cnt_sref[b], wait_one, jnp.int32(0))
        sl = pl.ds(b * QS, QS)
        acc_ref[sl] = jnp.where(
            maskf_ref[sl] > 0, stage_ref[sl], 0.0
        ).astype(jnp.bfloat16)

    hA = (my ^ (my >> 1)) & 1
    qA = my >> 1
    hB = my >> 1
    qB = my & 1

    khA = hA * HS
    shA = (1 - hA) * HS
    kqA = khA + qA * QS
    sqA = khA + (1 - qA) * QS
    khB = 2 * HS + hB * HS
    shB = 2 * HS + (1 - hB) * HS
    kqB = khB + qB * QS
    sqB = khB + (1 - qB) * QS

    def exch(sem_idx, src_off, n_rows, rbuf_off, peer):
        return pltpu.make_async_remote_copy(
            src_ref=acc_ref.at[pl.ds(src_off, n_rows)],
            dst_ref=rbuf_ref.at[pl.ds(rbuf_off, n_rows)],
            send_sem=send_sems.at[sem_idx], recv_sem=recv_sems.at[sem_idx],
            device_id=(peer,), device_id_type=pl.DeviceIdType.MESH,
        )

    bsA = shA // QS
    bsB = shB // QS
    bkA = khA // QS
    bkB = khB // QS

    gather_block(bsA, 0)
    drain_convert(bsA, 0)
    a1a = exch(0, shA, QS, 0, p1)
    a1a.start()

    gather_block(bsA + 1, 1)
    drain_convert(bsA + 1, 1)
    a1b = exch(1, shA + QS, QS, QS, p1)
    a1b.start()

    gather_block(bsB, 2)
    drain_convert(bsB, 2)
    b1a = exch(2, shB, QS, HS, p2)
    b1a.start()

    gather_block(bsB + 1, 3)
    drain_convert(bsB + 1, 3)
    b1b = exch(3, shB + QS, QS, HS + QS, p2)
    b1b.start()

    gather_block(bkA, 4)
    drain_convert(bkA, 4)
    gather_block(bkA + 1, 5)
    drain_convert(bkA + 1, 5)

    a1a.wait()
    acc_ref[pl.ds(khA, QS)] = acc_ref[pl.ds(khA, QS)] + rbuf_ref[pl.ds(0, QS)]
    a1b.wait()
    acc_ref[pl.ds(khA + QS, QS)] = (
        acc_ref[pl.ds(khA + QS, QS)] + rbuf_ref[pl.ds(QS, QS)]
    )
    a2 = exch(4, sqA, QS, 2 * HS, p2)
    a2.start()

    gather_block(bkB, 6)
    drain_convert(bkB, 6)
    gather_block(bkB + 1, 7)
    drain_convert(bkB + 1, 7)

    b1a.wait()
    acc_ref[pl.ds(khB, QS)] = acc_ref[pl.ds(khB, QS)] + rbuf_ref[pl.ds(HS, QS)]
    b1b.wait()
    acc_ref[pl.ds(khB + QS, QS)] = (
        acc_ref[pl.ds(khB + QS, QS)] + rbuf_ref[pl.ds(HS + QS, QS)]
    )
    b2 = exch(5, sqB, QS, 2 * HS + QS, p1)
    b2.start()

    a2.wait()
    acc_ref[pl.ds(kqA, QS)] = (
        acc_ref[pl.ds(kqA, QS)] + rbuf_ref[pl.ds(2 * HS, QS)]
    )
    a3 = exch(6, kqA, QS, 3 * HS, p2)
    a3.start()
    a4a = exch(8, kqA, QS, 4 * HS, p1)
    a4a.start()

    b2.wait()
    acc_ref[pl.ds(kqB, QS)] = (
        acc_ref[pl.ds(kqB, QS)] + rbuf_ref[pl.ds(2 * HS + QS, QS)]
    )
    b3 = exch(7, kqB, QS, 3 * HS + QS, p1)
    b3.start()
    b4a = exch(10, kqB, QS, 5 * HS, p2)
    b4a.start()

    a3.wait()
    acc_ref[pl.ds(sqA, QS)] = rbuf_ref[pl.ds(3 * HS, QS)]
    a4b = exch(9, sqA, QS, 4 * HS + QS, p1)
    a4b.start()

    b3.wait()
    acc_ref[pl.ds(sqB, QS)] = rbuf_ref[pl.ds(3 * HS + QS, QS)]
    b4b = exch(11, sqB, QS, 5 * HS + QS, p2)
    b4b.start()

    out_ref[pl.ds(khA, HS)] = acc_ref[pl.ds(khA, HS)].astype(jnp.float32)
    out_ref[pl.ds(khB, HS)] = acc_ref[pl.ds(khB, HS)].astype(jnp.float32)

    a4a.wait()
    out_ref[pl.ds(shA + qA * QS, QS)] = (
        rbuf_ref[pl.ds(4 * HS, QS)].astype(jnp.float32)
    )
    a4b.wait()
    out_ref[pl.ds(shA + (1 - qA) * QS, QS)] = (
        rbuf_ref[pl.ds(4 * HS + QS, QS)].astype(jnp.float32)
    )
    b4a.wait()
    out_ref[pl.ds(shB + (1 - qB) * QS, QS)] = (
        rbuf_ref[pl.ds(5 * HS, QS)].astype(jnp.float32)
    )
    b4b.wait()
    out_ref[pl.ds(shB + qB * QS, QS)] = (
        rbuf_ref[pl.ds(5 * HS + QS, QS)].astype(jnp.float32)
    )


def kernel(ids, E):
    my = lax.axis_index("i")
    loc = (ids - my * V_SHARD).astype(jnp.int32)
    owned = (loc >= 0) & (loc < V_SHARD)
    maskf = owned.astype(jnp.float32)[:, None]

    owned_b = owned.reshape(8, QS)
    loc_b = jnp.clip(loc, 0, V_SHARD - 1).reshape(8, QS)
    row_b = jnp.arange(T, dtype=jnp.int32).reshape(8, QS)
    cnt = owned_b.sum(axis=1).astype(jnp.int32)

    cs = jnp.cumsum(owned_b.astype(jnp.int32), axis=1)
    k = jnp.arange(NCB, dtype=jnp.int32)[None, :, None]
    M = ((cs[:, None, :] == (k + 1)) & owned_b[:, None, :]).astype(jnp.float32)
    R = jnp.stack([row_b, loc_b], axis=-1).astype(jnp.float32)
    pairs = jnp.matmul(M, R, precision=lax.Precision.HIGHEST).astype(jnp.int32)
    absrow = pairs[..., 0].reshape(-1)
    vid = pairs[..., 1].reshape(-1)

    grid_spec = pltpu.PrefetchScalarGridSpec(
        num_scalar_prefetch=3,
        grid=(1,),
        in_specs=[
            pl.BlockSpec(memory_space=pl.ANY),
            pl.BlockSpec(memory_space=pltpu.VMEM),
        ],
        out_specs=pl.BlockSpec(memory_space=pltpu.VMEM),
        scratch_shapes=[
            pltpu.VMEM((T, D), jnp.float32),
            pltpu.VMEM((T, D), jnp.bfloat16),
            pltpu.VMEM((6 * HS, D), jnp.bfloat16),
            pltpu.SemaphoreType.DMA((8,)),
            pltpu.SemaphoreType.DMA((12,)),
            pltpu.SemaphoreType.DMA((12,)),
        ],
    )
    return pl.pallas_call(
        _body,
        grid_spec=grid_spec,
        out_shape=jax.ShapeDtypeStruct((T, D), jnp.float32),
        compiler_params=pltpu.CompilerParams(collective_id=0),
    )(absrow, vid, cnt, E, maskf)
